# baseline (device time: 14606 ns/iter reference)
import jax
import jax.numpy as jnp
from jax import lax
from jax.experimental import pallas as pl
from jax.experimental.pallas import tpu as pltpu

N_DEV = 8
CAP = 12


def kernel(x, router_W, route_idx, expert_W):
    n, d = x.shape
    e_loc, _, h = expert_W.shape
    slots = e_loc * CAP

    def body(x_hbm, rw_hbm, idx_ref, w_hbm, out_ref, c_ref, comm_ref, x_v, w_v,
             send_sems, recv_sems, cp_sems):
        my = lax.axis_index("i")

        cp_x = pltpu.make_async_copy(x_hbm, x_v, cp_sems.at[0])
        cp_x.start()
        cp_w = pltpu.make_async_copy(w_hbm, w_v, cp_sems.at[1])
        cp_w.start()

        barrier_sem = pltpu.get_barrier_semaphore()
        for o in range(1, N_DEV):
            pl.semaphore_signal(
                barrier_sem, inc=1,
                device_id=(lax.rem(my + o, N_DEV),),
                device_id_type=pl.DeviceIdType.MESH,
            )

        route = idx_ref[:, :]
        n_exp = N_DEV * e_loc
        eids = lax.broadcasted_iota(jnp.int32, (n, n_exp), 1)
        onehot = (route == eids).astype(jnp.bfloat16)
        row = lax.broadcasted_iota(jnp.int32, (n, n), 0)
        col = lax.broadcasted_iota(jnp.int32, (n, n), 1)
        tri = (col < row).astype(jnp.bfloat16)
        excl = jnp.dot(tri, onehot, preferred_element_type=jnp.float32)
        rank = jnp.sum(onehot.astype(jnp.float32) * excl, axis=1, keepdims=True)

        jj = lax.broadcasted_iota(jnp.int32, (n, slots), 1)
        sel_e = my * e_loc + jj // CAP
        sel_c = (jj % CAP).astype(jnp.float32)
        S = ((route == sel_e) & (rank == sel_c)).astype(jnp.bfloat16)
        cp_x.wait()
        xg = lax.dot_general(
            S, x_v[:, :].astype(jnp.bfloat16), (((0,), (0,)), ((), ())),
            preferred_element_type=jnp.float32,
        )
        cp_w.wait()
        for e in range(e_loc):
            c_ref[e * CAP : (e + 1) * CAP, :] = jnp.dot(
                xg[e * CAP : (e + 1) * CAP, :].astype(jnp.bfloat16),
                w_v[e].astype(jnp.bfloat16),
                preferred_element_type=jnp.float32,
            ).astype(jnp.bfloat16)
        comm_ref[0, :, :] = c_ref[:, :]

        pl.semaphore_wait(barrier_sem, N_DEV - 1)

        rdmas = []
        for o in range(1, N_DEV):
            rdma = pltpu.make_async_remote_copy(
                src_ref=c_ref,
                dst_ref=comm_ref.at[o],
                send_sem=send_sems.at[o],
                recv_sem=recv_sems.at[o],
                device_id=(lax.rem(my + o, N_DEV),),
                device_id_type=pl.DeviceIdType.MESH,
            )
            rdma.start()
            rdmas.append(rdma)

        jall = lax.broadcasted_iota(jnp.int32, (n, N_DEV * slots), 1)
        src_all = lax.rem(my - jall // slots + N_DEV, N_DEV)
        e_all = src_all * e_loc + (jall % slots) // CAP
        c_all = (jall % CAP).astype(jnp.float32)
        P_all = ((route == e_all) & (rank == c_all)).astype(jnp.bfloat16)
        half = N_DEV // 2
        for o in range(1, half):
            rdmas[o - 1].wait_recv()
        acc = jnp.dot(
            P_all[:, : half * slots],
            comm_ref[:half, :, :].reshape(half * slots, h),
            preferred_element_type=jnp.float32,
        )
        for o in range(half, N_DEV):
            rdmas[o - 1].wait_recv()
        out_ref[:, :] = acc + jnp.dot(
            P_all[:, half * slots :],
            comm_ref[half:, :, :].reshape(half * slots, h),
            preferred_element_type=jnp.float32,
        )

        for o in range(1, N_DEV):
            rdmas[o - 1].wait_send()

    return pl.pallas_call(
        body,
        out_shape=jax.ShapeDtypeStruct((n, h), jnp.float32),
        in_specs=[
            pl.BlockSpec(memory_space=pl.ANY),
            pl.BlockSpec(memory_space=pl.ANY),
            pl.BlockSpec(memory_space=pltpu.VMEM),
            pl.BlockSpec(memory_space=pl.ANY),
        ],
        out_specs=pl.BlockSpec(memory_space=pltpu.VMEM),
        scratch_shapes=[
            pltpu.VMEM((slots, h), jnp.bfloat16),
            pltpu.VMEM((N_DEV, slots, h), jnp.bfloat16),
            pltpu.VMEM((n, d), jnp.float32),
            pltpu.VMEM((e_loc, d, h), jnp.float32),
            pltpu.SemaphoreType.DMA((N_DEV,)),
            pltpu.SemaphoreType.DMA((N_DEV,)),
            pltpu.SemaphoreType.DMA((2,)),
        ],
        compiler_params=pltpu.CompilerParams(collective_id=0),
    )(x, router_W, route_idx, expert_W)


# device time: 13617 ns/iter; 1.0726x vs baseline; 1.0726x over previous
import jax
import jax.numpy as jnp
from jax import lax
from jax.experimental import pallas as pl
from jax.experimental.pallas import tpu as pltpu

N_DEV = 8
CAP = 12


def kernel(x, router_W, route_idx, expert_W):
    n, d = x.shape
    e_loc, _, h = expert_W.shape
    slots = e_loc * CAP

    def body(x_ref, idx_ref, w_ref, out_ref, c_ref, comm_ref,
             send_sems, recv_sems):
        my = lax.axis_index("i")

        barrier_sem = pltpu.get_barrier_semaphore()
        for o in range(1, N_DEV):
            pl.semaphore_signal(
                barrier_sem, inc=1,
                device_id=(lax.rem(my + o, N_DEV),),
                device_id_type=pl.DeviceIdType.MESH,
            )

        route = idx_ref[:, :]
        n_exp = N_DEV * e_loc
        eids = lax.broadcasted_iota(jnp.int32, (n, n_exp), 1)
        onehot = (route == eids).astype(jnp.bfloat16)
        row = lax.broadcasted_iota(jnp.int32, (n, n), 0)
        col = lax.broadcasted_iota(jnp.int32, (n, n), 1)
        tri = (col < row).astype(jnp.bfloat16)
        excl = jnp.dot(tri, onehot, preferred_element_type=jnp.float32)
        rank = jnp.sum(onehot.astype(jnp.float32) * excl, axis=1, keepdims=True)

        jj = lax.broadcasted_iota(jnp.int32, (n, slots), 1)
        sel_e = my * e_loc + jj // CAP
        sel_c = (jj % CAP).astype(jnp.float32)
        S = ((route == sel_e) & (rank == sel_c)).astype(jnp.bfloat16)
        xg = lax.dot_general(
            S, x_ref[:, :], (((0,), (0,)), ((), ())),
            preferred_element_type=jnp.float32,
        )
        for e in range(e_loc):
            c_ref[e * CAP : (e + 1) * CAP, :] = jnp.dot(
                xg[e * CAP : (e + 1) * CAP, :].astype(jnp.bfloat16),
                w_ref[e],
                preferred_element_type=jnp.float32,
            ).astype(jnp.bfloat16)
        comm_ref[0, :, :] = c_ref[:, :]

        pl.semaphore_wait(barrier_sem, N_DEV - 1)

        rdmas = []
        for o in range(1, N_DEV):
            rdma = pltpu.make_async_remote_copy(
                src_ref=c_ref,
                dst_ref=comm_ref.at[o],
                send_sem=send_sems.at[o],
                recv_sem=recv_sems.at[o],
                device_id=(lax.rem(my + o, N_DEV),),
                device_id_type=pl.DeviceIdType.MESH,
            )
            rdma.start()
            rdmas.append(rdma)

        jall = lax.broadcasted_iota(jnp.int32, (n, N_DEV * slots), 1)
        src_all = lax.rem(my - jall // slots + N_DEV, N_DEV)
        e_all = src_all * e_loc + (jall % slots) // CAP
        c_all = (jall % CAP).astype(jnp.float32)
        P_all = ((route == e_all) & (rank == c_all)).astype(jnp.bfloat16)
        half = N_DEV // 2
        for o in range(1, half):
            rdmas[o - 1].wait_recv()
        acc = jnp.dot(
            P_all[:, : half * slots],
            comm_ref[:half, :, :].reshape(half * slots, h),
            preferred_element_type=jnp.float32,
        )
        for o in range(half, N_DEV):
            rdmas[o - 1].wait_recv()
        out_ref[:, :] = acc + jnp.dot(
            P_all[:, half * slots :],
            comm_ref[half:, :, :].reshape(half * slots, h),
            preferred_element_type=jnp.float32,
        )

        for o in range(1, N_DEV):
            rdmas[o - 1].wait_send()

    return pl.pallas_call(
        body,
        out_shape=jax.ShapeDtypeStruct((n, h), jnp.float32),
        in_specs=[
            pl.BlockSpec(memory_space=pltpu.VMEM),
            pl.BlockSpec(memory_space=pltpu.VMEM),
            pl.BlockSpec(memory_space=pltpu.VMEM),
        ],
        out_specs=pl.BlockSpec(memory_space=pltpu.VMEM),
        scratch_shapes=[
            pltpu.VMEM((slots, h), jnp.bfloat16),
            pltpu.VMEM((N_DEV, slots, h), jnp.bfloat16),
            pltpu.SemaphoreType.DMA((N_DEV,)),
            pltpu.SemaphoreType.DMA((N_DEV,)),
        ],
        compiler_params=pltpu.CompilerParams(collective_id=0),
    )(x.astype(jnp.bfloat16), route_idx, expert_W.astype(jnp.bfloat16))


# device time: 13202 ns/iter; 1.1063x vs baseline; 1.0314x over previous
import jax
import jax.numpy as jnp
from jax import lax
from jax.experimental import pallas as pl
from jax.experimental.pallas import tpu as pltpu

N_DEV = 8
CAP = 12


def kernel(x, router_W, route_idx, expert_W):
    n, d = x.shape
    e_loc, _, h = expert_W.shape
    slots = e_loc * CAP

    def body(x_ref, idx_ref, w_ref, out_ref, c_ref, comm_ref,
             send_sems, recv_sems):
        my = lax.axis_index("i")

        barrier_sem = pltpu.get_barrier_semaphore()
        for o in range(1, N_DEV):
            pl.semaphore_signal(
                barrier_sem, inc=1,
                device_id=(lax.rem(my + o, N_DEV),),
                device_id_type=pl.DeviceIdType.MESH,
            )

        route = idx_ref[:, :]
        n_exp = N_DEV * e_loc
        eids = lax.broadcasted_iota(jnp.int32, (n, n_exp), 1)
        onehot = (route == eids).astype(jnp.bfloat16)
        row = lax.broadcasted_iota(jnp.int32, (n, n), 0)
        col = lax.broadcasted_iota(jnp.int32, (n, n), 1)
        tri = (col < row).astype(jnp.bfloat16)
        excl = jnp.dot(tri, onehot, preferred_element_type=jnp.float32)
        rank = jnp.sum(onehot.astype(jnp.float32) * excl, axis=1, keepdims=True)

        jj = lax.broadcasted_iota(jnp.int32, (n, slots), 1)
        sel_e = my * e_loc + jj // CAP
        sel_c = (jj % CAP).astype(jnp.float32)
        S = ((route == sel_e) & (rank == sel_c)).astype(jnp.bfloat16)
        xg = lax.dot_general(
            S, x_ref[:, :].astype(jnp.bfloat16), (((0,), (0,)), ((), ())),
            preferred_element_type=jnp.float32,
        )
        pl.semaphore_wait(barrier_sem, N_DEV - 1)

        for e in range(e_loc):
            c_ref[e * CAP : (e + 1) * CAP, :] = jnp.dot(
                xg[e * CAP : (e + 1) * CAP, :].astype(jnp.bfloat16),
                w_ref[e].astype(jnp.bfloat16),
                preferred_element_type=jnp.float32,
            ).astype(jnp.bfloat16)
        comm_ref[0, :, :] = c_ref[:, :]

        rdmas = []
        for o in range(1, N_DEV):
            rdma = pltpu.make_async_remote_copy(
                src_ref=c_ref,
                dst_ref=comm_ref.at[o],
                send_sem=send_sems.at[o],
                recv_sem=recv_sems.at[o],
                device_id=(lax.rem(my + o, N_DEV),),
                device_id_type=pl.DeviceIdType.MESH,
            )
            rdma.start()
            rdmas.append(rdma)

        jall = lax.broadcasted_iota(jnp.int32, (n, N_DEV * slots), 1)
        src_all = lax.rem(my - jall // slots + N_DEV, N_DEV)
        e_all = src_all * e_loc + (jall % slots) // CAP
        c_all = (jall % CAP).astype(jnp.float32)
        P_all = ((route == e_all) & (rank == c_all)).astype(jnp.bfloat16)
        half = N_DEV // 2
        for o in range(1, half):
            rdmas[o - 1].wait_recv()
        acc = jnp.dot(
            P_all[:, : half * slots],
            comm_ref[:half, :, :].reshape(half * slots, h),
            preferred_element_type=jnp.float32,
        )
        for o in range(half, N_DEV):
            rdmas[o - 1].wait_recv()
        out_ref[:, :] = acc + jnp.dot(
            P_all[:, half * slots :],
            comm_ref[half:, :, :].reshape(half * slots, h),
            preferred_element_type=jnp.float32,
        )

        for r in rdmas:
            r.wait_send()

    return pl.pallas_call(
        body,
        out_shape=jax.ShapeDtypeStruct((n, h), jnp.float32),
        in_specs=[
            pl.BlockSpec(memory_space=pltpu.VMEM),
            pl.BlockSpec(memory_space=pltpu.VMEM),
            pl.BlockSpec(memory_space=pltpu.VMEM),
        ],
        out_specs=pl.BlockSpec(memory_space=pltpu.VMEM),
        scratch_shapes=[
            pltpu.VMEM((slots, h), jnp.bfloat16),
            pltpu.VMEM((N_DEV, slots, h), jnp.bfloat16),
            pltpu.SemaphoreType.DMA((N_DEV,)),
            pltpu.SemaphoreType.DMA((N_DEV,)),
        ],
        compiler_params=pltpu.CompilerParams(collective_id=0),
    )(x, route_idx, expert_W)
